# baseline (speedup 1.0000x reference)
import functools

import jax
import jax.numpy as jnp
from jax import lax
from jax.experimental import pallas as pl
from jax.experimental.pallas import tpu as pltpu
from jax.experimental.pallas import tpu_sc as plsc

BATCH = 16384
EMBED = 64

_NC, _NS, _L = 2, 16, 16
_NW = _NC * _NS
_BPW = BATCH // _NW
_CHUNK = 128
_NCHUNK = _BPW // _CHUNK
_BLOCKS = _CHUNK // 16

_REV = (0, 8, 4, 12, 2, 10, 6, 14, 1, 9, 5, 13, 3, 11, 7, 15)


def _sc_body(user_hbm, movie_hbm, utab_hbm, mtab_hbm, w_hbm, b_hbm, out_hbm,
             uidx, midx, urows, mrows, wv, bv, outv, sem):
    wid = lax.axis_index("s") * _NC + lax.axis_index("c")
    base = wid * _BPW

    pltpu.sync_copy(w_hbm, wv)
    pltpu.sync_copy(b_hbm, bv)
    for j in range(_NCHUNK):
        pltpu.sync_copy(user_hbm.at[pl.ds(base + j * _CHUNK, _CHUNK)],
                        uidx.at[j])
        pltpu.sync_copy(movie_hbm.at[pl.ds(base + j * _CHUNK, _CHUNK)],
                        midx.at[j])

    copies = []
    for j in range(_NCHUNK):
        copies.append(pltpu.async_copy(utab_hbm.at[uidx.at[j]], urows.at[j],
                                       sem))
        copies.append(pltpu.async_copy(mtab_hbm.at[midx.at[j]], mrows.at[j],
                                       sem))

    wu = [wv[pl.ds(16 * c, 16)] for c in range(4)]
    wm = [wv[pl.ds(64 + 16 * c, 16)] for c in range(4)]
    bias = bv[...]
    ii = lax.iota(jnp.int32, 16)
    masks = {d: (ii & d) == 0 for d in (8, 4, 2, 1)}
    perms = {d: ii ^ d for d in (8, 4, 2, 1)}

    gdn = lax.GatherDimensionNumbers(offset_dims=(), collapsed_slice_dims=(0,),
                                     start_index_map=(0,))

    def permute(v, p):
        return lax.gather(v, p[:, None], gdn, slice_sizes=(1,),
                          mode=lax.GatherScatterMode.PROMISE_IN_BOUNDS)

    def combine(a, b, d):
        lo = jnp.where(masks[d], a, b)
        hi = jnp.where(masks[d], b, a)
        return lo + permute(hi, perms[d])

    for j in range(_NCHUNK):
        copies[2 * j].wait()
        copies[2 * j + 1].wait()

        def block_body(b, carry, j=j):
            row0 = b * 16
            s = []
            for r in range(16):
                rr = row0 + _REV[r]
                acc = urows[j, rr, pl.ds(0, 16)] * wu[0]
                for c in range(1, 4):
                    acc = acc + urows[j, rr, pl.ds(16 * c, 16)] * wu[c]
                for c in range(4):
                    acc = acc + mrows[j, rr, pl.ds(16 * c, 16)] * wm[c]
                s.append(acc)
            for d in (8, 4, 2, 1):
                s = [combine(s[2 * i], s[2 * i + 1], d)
                     for i in range(len(s) // 2)]
            outv[pl.ds(j * _CHUNK + row0, 16)] = s[0] + bias
            return carry

        lax.fori_loop(0, _BLOCKS, block_body, 0)

    pltpu.sync_copy(outv, out_hbm.at[pl.ds(base, _BPW)])


@jax.jit
def _sc_call(user, movie, user_table, movie_table, w_vec, b_vec):
    mesh = plsc.VectorSubcoreMesh(core_axis_name="c", subcore_axis_name="s")
    fn = pl.kernel(
        _sc_body,
        mesh=mesh,
        compiler_params=pltpu.CompilerParams(use_tc_tiling_on_sc=False),
        out_type=jax.ShapeDtypeStruct((BATCH,), jnp.float32),
        scratch_types=[
            pltpu.VMEM((_NCHUNK, _CHUNK), jnp.int32),
            pltpu.VMEM((_NCHUNK, _CHUNK), jnp.int32),
            pltpu.VMEM((_NCHUNK, _CHUNK, EMBED), jnp.float32),
            pltpu.VMEM((_NCHUNK, _CHUNK, EMBED), jnp.float32),
            pltpu.VMEM((2 * EMBED,), jnp.float32),
            pltpu.VMEM((16,), jnp.float32),
            pltpu.VMEM((_BPW,), jnp.float32),
            pltpu.SemaphoreType.DMA,
        ],
    )
    return fn(user, movie, user_table, movie_table, w_vec, b_vec)


def kernel(user, movie, user_table, movie_table, fc_w, fc_b):
    w_vec = fc_w.reshape(2 * EMBED).astype(jnp.float32)
    b_vec = jnp.broadcast_to(fc_b.astype(jnp.float32), (16,))
    out = _sc_call(user.astype(jnp.int32), movie.astype(jnp.int32),
                   user_table, movie_table, w_vec, b_vec)
    return out.reshape(BATCH, 1)


# speedup vs baseline: 5.3897x; 5.3897x over previous
import functools

import jax
import jax.numpy as jnp
from jax import lax
from jax.experimental import pallas as pl
from jax.experimental.pallas import tpu as pltpu
from jax.experimental.pallas import tpu_sc as plsc

BATCH = 16384
EMBED = 64
NUSER = 1000000
NMOVIE = 100000

_NC, _NS, _L = 2, 16, 16
_NW = _NC * _NS
_BPW = BATCH // _NW
_CHUNK = 128
_NCHUNK = _BPW // _CHUNK
_BLOCKS = _CHUNK // 16

_TC_BN = 16384


def _tc_dot_body(w_ref, x_ref, o_ref):
    o_ref[...] = jnp.dot(w_ref[...], x_ref[...],
                         preferred_element_type=jnp.float32)[0]


def _tc_dot(w_half, table_t, n):
    grid = (n + _TC_BN - 1) // _TC_BN
    return pl.pallas_call(
        _tc_dot_body,
        grid=(grid,),
        in_specs=[
            pl.BlockSpec((1, EMBED), lambda i: (0, 0)),
            pl.BlockSpec((EMBED, _TC_BN), lambda i: (0, i)),
        ],
        out_specs=pl.BlockSpec((_TC_BN,), lambda i: (i,)),
        out_shape=jax.ShapeDtypeStruct((n,), jnp.float32),
    )(w_half, table_t)


def _sc_gather_body(user_hbm, movie_hbm, pu_hbm, pm_hbm, b_hbm, out_hbm,
                    uidx, midx, urid, mrid, urows, mrows, bv, outv, sem):
    wid = lax.axis_index("s") * _NC + lax.axis_index("c")
    base = wid * _BPW

    pltpu.sync_copy(b_hbm, bv)
    for j in range(_NCHUNK):
        pltpu.sync_copy(user_hbm.at[pl.ds(base + j * _CHUNK, _CHUNK)],
                        uidx.at[j])
        pltpu.sync_copy(movie_hbm.at[pl.ds(base + j * _CHUNK, _CHUNK)],
                        midx.at[j])

    for j in range(_NCHUNK):
        for b in range(_BLOCKS):
            s = pl.ds(16 * b, 16)
            urid[j, s] = lax.shift_right_logical(uidx[j, s], 4)
            mrid[j, s] = lax.shift_right_logical(midx[j, s], 4)

    copies = []
    for j in range(_NCHUNK):
        copies.append(pltpu.async_copy(pu_hbm.at[urid.at[j]], urows.at[j],
                                       sem.at[j]))
        copies.append(pltpu.async_copy(pm_hbm.at[mrid.at[j]], mrows.at[j],
                                       sem.at[j]))

    bias = bv[...]
    ii = lax.iota(jnp.int32, 16)
    mask15 = jnp.full((16,), 15, dtype=jnp.int32)

    for j in range(_NCHUNK):
        copies[2 * j].wait()
        copies[2 * j + 1].wait()
        for b in range(_BLOCKS):
            s = pl.ds(16 * b, 16)
            row = ii + (16 * b)
            gu = plsc.load_gather(urows.at[j], [row, uidx[j, s] & mask15])
            gm = plsc.load_gather(mrows.at[j], [row, midx[j, s] & mask15])
            outv[pl.ds(j * _CHUNK + 16 * b, 16)] = gu + gm + bias

    pltpu.sync_copy(outv, out_hbm.at[pl.ds(base, _BPW)])


@jax.jit
def _call(user, movie, user_table, movie_table, w_vec, b_vec):
    pu = _tc_dot(w_vec[:EMBED].reshape(1, EMBED), user_table.T, NUSER)
    pm = _tc_dot(w_vec[EMBED:].reshape(1, EMBED), movie_table.T, NMOVIE)
    pu2 = pu.reshape(NUSER // 16, 16)
    pm2 = pm.reshape(NMOVIE // 16, 16)

    mesh = plsc.VectorSubcoreMesh(core_axis_name="c", subcore_axis_name="s")
    fn = pl.kernel(
        _sc_gather_body,
        mesh=mesh,
        compiler_params=pltpu.CompilerParams(use_tc_tiling_on_sc=False,
                                             needs_layout_passes=False),
        out_type=jax.ShapeDtypeStruct((BATCH,), jnp.float32),
        scratch_types=[
            pltpu.VMEM((_NCHUNK, _CHUNK), jnp.int32),
            pltpu.VMEM((_NCHUNK, _CHUNK), jnp.int32),
            pltpu.VMEM((_NCHUNK, _CHUNK), jnp.int32),
            pltpu.VMEM((_NCHUNK, _CHUNK), jnp.int32),
            pltpu.VMEM((_NCHUNK, _CHUNK, 16), jnp.float32),
            pltpu.VMEM((_NCHUNK, _CHUNK, 16), jnp.float32),
            pltpu.VMEM((16,), jnp.float32),
            pltpu.VMEM((_BPW,), jnp.float32),
            pltpu.SemaphoreType.DMA((_NCHUNK,)),
        ],
    )
    return fn(user, movie, pu2, pm2, b_vec)


def kernel(user, movie, user_table, movie_table, fc_w, fc_b):
    w_vec = fc_w.reshape(2 * EMBED).astype(jnp.float32)
    b_vec = jnp.broadcast_to(fc_b.astype(jnp.float32), (16,))
    out = _call(user.astype(jnp.int32), movie.astype(jnp.int32),
                user_table, movie_table, w_vec, b_vec)
    return out.reshape(BATCH, 1)
